# baseline (speedup 1.0000x reference)
import functools

import jax
import jax.numpy as jnp
from jax import lax
from jax.experimental import pallas as pl
from jax.experimental.pallas import tpu as pltpu
from jax.experimental.pallas import tpu_sc as plsc

SEQ = 2048
HIDDEN = 64
VOCAB = 512
B = SEQ * SEQ
NW = 32
LOOK_PER_W = B // NW
CH = 512
NG = LOOK_PER_W // CH
NBUF = 2
L = 16


def _make_gather():
    mesh = plsc.VectorSubcoreMesh(core_axis_name="c", subcore_axis_name="s")

    @functools.partial(
        pl.kernel,
        mesh=mesh,
        out_type=jax.ShapeDtypeStruct((B * HIDDEN,), jnp.float32),
        scratch_types=[
            pltpu.VMEM((VOCAB * HIDDEN,), jnp.float32),
            pltpu.VMEM((CH,), jnp.int32),
            pltpu.VMEM((CH,), jnp.int32),
            pltpu.VMEM((CH * HIDDEN,), jnp.float32),
            pltpu.VMEM((CH * HIDDEN,), jnp.float32),
            pltpu.SemaphoreType.DMA,
            pltpu.SemaphoreType.DMA,
            pltpu.SemaphoreType.DMA,
            pltpu.SemaphoreType.DMA,
        ],
        compiler_params=pltpu.CompilerParams(
            use_tc_tiling_on_sc=False, needs_layout_passes=False),
    )
    def gather_kernel(table_hbm, idx_hbm, out_hbm,
                      table_v, idx_v0, idx_v1, stage_v0, stage_v1,
                      si0, si1, so0, so1):
        idx_bufs = (idx_v0, idx_v1)
        stage_bufs = (stage_v0, stage_v1)
        sem_i = (si0, si1)
        sem_o = (so0, so1)

        c = lax.axis_index("c")
        s = lax.axis_index("s")
        wid = s * 2 + c
        base_look = wid * LOOK_PER_W

        pltpu.sync_copy(table_hbm, table_v)

        def idx_start(g, p):
            pltpu.async_copy(
                idx_hbm.at[pl.ds(base_look + g * CH, CH)],
                idx_bufs[p], sem_i[p])

        def scatter_desc(g, p):
            return pltpu.make_async_copy(
                stage_bufs[p],
                out_hbm.at[pl.ds((base_look + g * CH) * HIDDEN, CH * HIDDEN)],
                sem_o[p])

        idx_start(0, 0)

        lane_iota = lax.iota(jnp.int32, L)
        dst_iota = lane_iota * HIDDEN

        def group(g, p):
            @pl.when(g >= NBUF)
            def _():
                scatter_desc(g - NBUF, p).wait()

            pltpu.make_async_copy(
                idx_hbm.at[pl.ds(0, CH)], idx_bufs[p], sem_i[p]).wait()

            @pl.when(g + 1 < NG)
            def _():
                idx_start(g + 1, 1 - p)

            def b_body(b, carry):
                idx16 = idx_bufs[p][pl.ds(b * L, L)]
                src_base = idx16 * HIDDEN
                dst_base = dst_iota + b * (L * HIDDEN)
                for col in range(HIDDEN):
                    vals = plsc.load_gather(table_v, [src_base + col])
                    plsc.store_scatter(stage_bufs[p], [dst_base + col], vals)
                return carry

            lax.fori_loop(0, CH // L, b_body, 0)

            pltpu.async_copy(
                stage_bufs[p],
                out_hbm.at[pl.ds((base_look + g * CH) * HIDDEN, CH * HIDDEN)],
                sem_o[p])

        def outer(gg, carry):
            for p in range(NBUF):
                group(gg * NBUF + p, p)
            return carry

        lax.fori_loop(0, NG // NBUF, outer, 0)

        for p in range(NBUF):
            scatter_desc(NG - NBUF + p, p).wait()

    return gather_kernel


_gather = _make_gather()


def kernel(dist_mat, table):
    idx = dist_mat.astype(jnp.int32).reshape(B)
    out = _gather(table.reshape(VOCAB * HIDDEN), idx)
    return out.reshape(SEQ, SEQ, HIDDEN)


# speedup vs baseline: 1.1474x; 1.1474x over previous
import functools

import jax
import jax.numpy as jnp
from jax import lax
from jax.experimental import pallas as pl
from jax.experimental.pallas import tpu as pltpu
from jax.experimental.pallas import tpu_sc as plsc

SEQ = 2048
HIDDEN = 64
VOCAB = 512
B = SEQ * SEQ
NW = 32
LOOK_PER_W = B // NW
CH = 512
NG = LOOK_PER_W // CH
NBUF = 2
L = 16


def _make_gather():
    mesh = plsc.VectorSubcoreMesh(core_axis_name="c", subcore_axis_name="s")

    @functools.partial(
        pl.kernel,
        mesh=mesh,
        out_type=jax.ShapeDtypeStruct((B * HIDDEN,), jnp.float32),
        scratch_types=[
            pltpu.VMEM((VOCAB * HIDDEN,), jnp.float32),
            pltpu.VMEM((CH,), jnp.int32),
            pltpu.VMEM((CH,), jnp.int32),
            pltpu.VMEM((CH * HIDDEN,), jnp.float32),
            pltpu.VMEM((CH * HIDDEN,), jnp.float32),
            pltpu.SemaphoreType.DMA,
            pltpu.SemaphoreType.DMA,
            pltpu.SemaphoreType.DMA,
            pltpu.SemaphoreType.DMA,
        ],
        compiler_params=pltpu.CompilerParams(
            use_tc_tiling_on_sc=False, needs_layout_passes=False),
    )
    def gather_kernel(table_hbm, idx_hbm, out_hbm,
                      table_v, idx_v0, idx_v1, stage_v0, stage_v1,
                      si0, si1, so0, so1):
        idx_bufs = (idx_v0, idx_v1)
        stage_bufs = (stage_v0, stage_v1)
        sem_i = (si0, si1)
        sem_o = (so0, so1)

        c = lax.axis_index("c")
        s = lax.axis_index("s")
        wid = s * 2 + c
        base_look = wid * LOOK_PER_W

        pltpu.sync_copy(table_hbm, table_v)

        def idx_start(g, p):
            pltpu.async_copy(
                idx_hbm.at[pl.ds(base_look + g * CH, CH)],
                idx_bufs[p], sem_i[p])

        def scatter_desc(g, p):
            return pltpu.make_async_copy(
                stage_bufs[p],
                out_hbm.at[pl.ds((base_look + g * CH) * HIDDEN, CH * HIDDEN)],
                sem_o[p])

        idx_start(0, 0)

        lane_iota = lax.iota(jnp.int32, L)
        dst_iota = lane_iota * HIDDEN

        def group(g, p):
            @pl.when(g >= NBUF)
            def _():
                scatter_desc(g - NBUF, p).wait()

            pltpu.make_async_copy(
                idx_hbm.at[pl.ds(0, CH)], idx_bufs[p], sem_i[p]).wait()

            @pl.when(g + 1 < NG)
            def _():
                idx_start(g + 1, 1 - p)

            @plsc.parallel_loop(0, CH // L, unroll=2)
            def b_body(b):
                idx16 = idx_bufs[p][pl.ds(b * L, L)]
                src_base = idx16 * HIDDEN
                dst_base = dst_iota + b * (L * HIDDEN)
                for col in range(HIDDEN):
                    vals = plsc.load_gather(table_v, [src_base + col])
                    plsc.store_scatter(stage_bufs[p], [dst_base + col], vals)

            pltpu.async_copy(
                stage_bufs[p],
                out_hbm.at[pl.ds((base_look + g * CH) * HIDDEN, CH * HIDDEN)],
                sem_o[p])

        def outer(gg, carry):
            for p in range(NBUF):
                group(gg * NBUF + p, p)
            return carry

        lax.fori_loop(0, NG // NBUF, outer, 0)

        for p in range(NBUF):
            scatter_desc(NG - NBUF + p, p).wait()

    return gather_kernel


_gather = _make_gather()


def kernel(dist_mat, table):
    idx = dist_mat.astype(jnp.int32).reshape(B)
    out = _gather(table.reshape(VOCAB * HIDDEN), idx)
    return out.reshape(SEQ, SEQ, HIDDEN)


# speedup vs baseline: 3.1123x; 2.7125x over previous
import functools

import jax
import jax.numpy as jnp
from jax import lax
from jax.experimental import pallas as pl
from jax.experimental.pallas import tpu as pltpu
from jax.experimental.pallas import tpu_sc as plsc

SEQ = 2048
HIDDEN = 64
VOCAB = 512
B = SEQ * SEQ
NW = 32
LOOK_PER_W = B // NW
CH = 512
NG = LOOK_PER_W // CH
NBUF = 2
L = 16


def _make_gather():
    mesh = plsc.VectorSubcoreMesh(core_axis_name="c", subcore_axis_name="s")

    @functools.partial(
        pl.kernel,
        mesh=mesh,
        out_type=jax.ShapeDtypeStruct((B * HIDDEN,), jnp.float32),
        scratch_types=[
            pltpu.VMEM((VOCAB * HIDDEN,), jnp.float32),
            pltpu.VMEM((CH,), jnp.int32),
            pltpu.VMEM((CH,), jnp.int32),
            pltpu.VMEM((CH * HIDDEN,), jnp.float32),
            pltpu.VMEM((CH * HIDDEN,), jnp.float32),
            pltpu.SemaphoreType.DMA,
            pltpu.SemaphoreType.DMA,
            pltpu.SemaphoreType.DMA,
            pltpu.SemaphoreType.DMA,
        ],
        compiler_params=pltpu.CompilerParams(
            use_tc_tiling_on_sc=False, needs_layout_passes=False),
    )
    def gather_kernel(table_hbm, idx_hbm, out_hbm,
                      table_v, idx_v0, idx_v1, stage_v0, stage_v1,
                      si0, si1, so0, so1):
        idx_bufs = (idx_v0, idx_v1)
        stage_bufs = (stage_v0, stage_v1)
        sem_i = (si0, si1)
        sem_o = (so0, so1)

        c = lax.axis_index("c")
        s = lax.axis_index("s")
        wid = s * 2 + c
        base_look = wid * LOOK_PER_W

        pltpu.sync_copy(table_hbm, table_v)

        def idx_start(g, p):
            pltpu.async_copy(
                idx_hbm.at[pl.ds(base_look + g * CH, CH)],
                idx_bufs[p], sem_i[p])

        def scatter_desc(g, p):
            return pltpu.make_async_copy(
                stage_bufs[p],
                out_hbm.at[pl.ds((base_look + g * CH) * HIDDEN, CH * HIDDEN)],
                sem_o[p])

        idx_start(0, 0)

        lane_iota = lax.iota(jnp.int32, L)
        dst_iota = lane_iota * HIDDEN

        def group(g, p):
            @pl.when(g >= NBUF)
            def _():
                scatter_desc(g - NBUF, p).wait()

            pltpu.make_async_copy(
                idx_hbm.at[pl.ds(0, CH)], idx_bufs[p], sem_i[p]).wait()

            @pl.when(g + 1 < NG)
            def _():
                idx_start(g + 1, 1 - p)

            @plsc.parallel_loop(0, CH // L, unroll=2)
            def b_body(b):
                idx16 = idx_bufs[p][pl.ds(b * L, L)]
                src_base = idx16 * HIDDEN
                dst_base = dst_iota + b * (L * HIDDEN)
                for col in range(HIDDEN):
                    sk = (lane_iota + col) & (HIDDEN - 1)
                    vals = plsc.load_gather(table_v, [src_base + sk])
                    plsc.store_scatter(stage_bufs[p], [dst_base + sk], vals)

            pltpu.async_copy(
                stage_bufs[p],
                out_hbm.at[pl.ds((base_look + g * CH) * HIDDEN, CH * HIDDEN)],
                sem_o[p])

        def outer(gg, carry):
            for p in range(NBUF):
                group(gg * NBUF + p, p)
            return carry

        lax.fori_loop(0, NG // NBUF, outer, 0)

        for p in range(NBUF):
            scatter_desc(NG - NBUF + p, p).wait()

    return gather_kernel


_gather = _make_gather()


def kernel(dist_mat, table):
    idx = dist_mat.astype(jnp.int32).reshape(B)
    out = _gather(table.reshape(VOCAB * HIDDEN), idx)
    return out.reshape(SEQ, SEQ, HIDDEN)
